# baseline (speedup 1.0000x reference)
import functools

import jax
import jax.numpy as jnp
from jax import lax
from jax.experimental import pallas as pl
from jax.experimental.pallas import tpu as pltpu
from jax.experimental.pallas import tpu_sc as plsc

N = 10000
D = 128
E = 320000

NC = 2
NS = 16
NW = NC * NS
K = 128
G = 8
NCHF = 160
NCHS = 0
NGF = NCHF // G
NGS = NCHS // G
E_PAD = NS * (NCHF + NCHS) * K
N_PAD = 10112
ROWS_PER_TILE = N_PAD // NS

_BLK = 2000


def _seg_sum_body(h_hbm, sd_hbm, out_hbm,
                  ib0, ib1, buf0, buf1, isem0, isem1, dsem0, dsem1, acc):
    c = lax.axis_index("c")
    s = lax.axis_index("s")
    wid = c * NS + s
    ng = jnp.where(c == 0, NGF, NGS)

    @pl.when(ng > 0)
    def _():
        pltpu.async_copy(sd_hbm.at[wid, pl.ds(0, G)], ib0, isem0)

    zero = jnp.zeros((16,), jnp.float32)

    def _zrow(i, carry):
        for l in range(D // 16):
            buf0[i, pl.ds(l * 16, 16)] = zero
        return carry

    lax.fori_loop(0, K, _zrow, 0)
    for z in range(ROWS_PER_TILE // K):
        pltpu.sync_copy(buf0, acc.at[pl.ds(s * ROWS_PER_TILE + z * K, K)])
    _rem = ROWS_PER_TILE % K
    if _rem:
        pltpu.sync_copy(
            buf0.at[pl.ds(0, _rem)],
            acc.at[pl.ds(s * ROWS_PER_TILE + (ROWS_PER_TILE // K) * K, _rem)])
    plsc.subcore_barrier()

    def _process(g, ib_cur, isem_cur, ib_next, isem_next):
        pltpu.make_async_copy(sd_hbm.at[wid, pl.ds(0, G)], ib_cur,
                              isem_cur).wait()

        @pl.when(g + 1 < ng)
        def _():
            pltpu.async_copy(sd_hbm.at[wid, pl.ds((g + 1) * G, G)],
                             ib_next, isem_next)

        pltpu.async_copy(h_hbm.at[ib_cur.at[0, 0]], buf0, dsem0)
        pltpu.async_copy(h_hbm.at[ib_cur.at[1, 0]], buf1, dsem1)
        for i in range(G):
            buf, dsem = (buf0, dsem0) if i % 2 == 0 else (buf1, dsem1)
            pltpu.make_async_copy(h_hbm.at[ib_cur.at[i, 0]], buf,
                                  dsem).wait()
            pltpu.sync_copy(buf, acc.at[ib_cur.at[i, 1]], add=True)
            if i + 2 < G:
                pltpu.async_copy(h_hbm.at[ib_cur.at[i + 2, 0]], buf, dsem)

    def _body(g, carry):
        @pl.when(g % 2 == 0)
        def _():
            _process(g, ib0, isem0, ib1, isem1)

        @pl.when(g % 2 == 1)
        def _():
            _process(g, ib1, isem1, ib0, isem0)

        return carry

    lax.fori_loop(0, ng, _body, 0)
    plsc.subcore_barrier()

    pltpu.sync_copy(acc.at[pl.ds(s * ROWS_PER_TILE, ROWS_PER_TILE)],
                    out_hbm.at[c, pl.ds(s * ROWS_PER_TILE, ROWS_PER_TILE)])


_seg_sum = functools.partial(
    pl.kernel,
    out_type=jax.ShapeDtypeStruct((NC, N_PAD, D), jnp.float32),
    mesh=plsc.VectorSubcoreMesh(core_axis_name="c", subcore_axis_name="s"),
    scratch_types=[
        pltpu.VMEM((G, 2, K), jnp.int32),
        pltpu.VMEM((G, 2, K), jnp.int32),
        pltpu.VMEM((K, D), jnp.float32),
        pltpu.VMEM((K, D), jnp.float32),
        pltpu.SemaphoreType.DMA,
        pltpu.SemaphoreType.DMA,
        pltpu.SemaphoreType.DMA,
        pltpu.SemaphoreType.DMA,
        pltpu.VMEM_SHARED((N_PAD, D), jnp.float32),
    ],
)(_seg_sum_body)


def _pack_edges(edge_index):
    pad = E_PAD - E
    src = jnp.concatenate([edge_index[0], jnp.zeros((pad,), jnp.int32)])
    dst = jnp.concatenate([edge_index[1], jnp.full((pad,), N, jnp.int32)])
    sd = jnp.stack([src.reshape(-1, K), dst.reshape(-1, K)], axis=1)
    nf = NS * NCHF
    fast = sd[:nf].reshape(NS, NCHF, 2, K)
    slow = sd[nf:].reshape(NS, NCHS, 2, K)
    slow = jnp.pad(slow, ((0, 0), (0, NCHF - NCHS), (0, 0), (0, 0)))
    return jnp.concatenate([fast, slow], axis=0)


def _dense0_body(x_ref, w_ref, b_ref, o_ref):
    o_ref[...] = (jnp.dot(x_ref[...], w_ref[...],
                          preferred_element_type=jnp.float32) + b_ref[...])


def _dense0(x, Wx, bx):
    return pl.pallas_call(
        _dense0_body,
        grid=(N // _BLK,),
        in_specs=[
            pl.BlockSpec((_BLK, D), lambda i: (i, 0)),
            pl.BlockSpec((D, D), lambda i: (0, 0)),
            pl.BlockSpec((1, D), lambda i: (0, 0)),
        ],
        out_specs=pl.BlockSpec((_BLK, D), lambda i: (i, 0)),
        out_shape=jax.ShapeDtypeStruct((N, D), jnp.float32),
    )(x, Wx, bx.reshape(1, D))


def _layer_a_body(p0_ref, p1_ref, h_ref, wrel_ref, wroot_ref, brel_ref,
                  t_ref, s1_ref, s2_ref):
    agg = p0_ref[...] + p1_ref[...]
    h = h_ref[...]
    t = (jnp.dot(agg, wrel_ref[...], preferred_element_type=jnp.float32)
         + jnp.dot(h, wroot_ref[...], preferred_element_type=jnp.float32)
         + h + brel_ref[...])
    t_ref[...] = t

    @pl.when(pl.program_id(0) == 0)
    def _():
        s1_ref[...] = jnp.zeros_like(s1_ref)
        s2_ref[...] = jnp.zeros_like(s2_ref)

    s1_ref[...] += jnp.broadcast_to(jnp.sum(t, axis=0), (8, D))
    s2_ref[...] += jnp.broadcast_to(jnp.sum(t * t, axis=0), (8, D))


def _layer_a(p0, p1, h, wrel, wroot, brel):
    return pl.pallas_call(
        _layer_a_body,
        grid=(N // _BLK,),
        in_specs=[
            pl.BlockSpec((_BLK, D), lambda i: (i, 0)),
            pl.BlockSpec((_BLK, D), lambda i: (i, 0)),
            pl.BlockSpec((_BLK, D), lambda i: (i, 0)),
            pl.BlockSpec((D, D), lambda i: (0, 0)),
            pl.BlockSpec((D, D), lambda i: (0, 0)),
            pl.BlockSpec((1, D), lambda i: (0, 0)),
        ],
        out_specs=[
            pl.BlockSpec((_BLK, D), lambda i: (i, 0)),
            pl.BlockSpec((8, D), lambda i: (0, 0)),
            pl.BlockSpec((8, D), lambda i: (0, 0)),
        ],
        out_shape=[
            jax.ShapeDtypeStruct((N, D), jnp.float32),
            jax.ShapeDtypeStruct((8, D), jnp.float32),
            jax.ShapeDtypeStruct((8, D), jnp.float32),
        ],
    )(p0, p1, h, wrel, wroot, brel)


def _layer_b_body(t_ref, s1_ref, s2_ref, g_ref, bb_ref, o_ref, *, relu):
    mu = s1_ref[0:1, :] * (1.0 / N)
    var = s2_ref[0:1, :] * (1.0 / N) - mu * mu
    inv = lax.rsqrt(var + 1e-5)
    o = g_ref[...] * ((t_ref[...] - mu) * inv) + bb_ref[...]
    if relu:
        o = jnp.maximum(o, 0.0)
    o_ref[...] = o


def _layer_b(t, s1, s2, gamma, beta, relu):
    return pl.pallas_call(
        functools.partial(_layer_b_body, relu=relu),
        grid=(N // _BLK,),
        in_specs=[
            pl.BlockSpec((_BLK, D), lambda i: (i, 0)),
            pl.BlockSpec((8, D), lambda i: (0, 0)),
            pl.BlockSpec((8, D), lambda i: (0, 0)),
            pl.BlockSpec((1, D), lambda i: (0, 0)),
            pl.BlockSpec((1, D), lambda i: (0, 0)),
        ],
        out_specs=pl.BlockSpec((_BLK, D), lambda i: (i, 0)),
        out_shape=jax.ShapeDtypeStruct((N, D), jnp.float32),
    )(t, s1, s2, gamma.reshape(1, D), beta.reshape(1, D))


def kernel(x, edge_index, edge_attr, edge_index_full, Wx, bx,
           Wrel0, brel0, Wroot0, Wrel1, brel1, Wroot1,
           Wrel1_2h, brel1_2h, Wroot1_2h, gamma0, beta0, gamma1, beta1):
    h0 = _dense0(x, Wx, bx)

    sd0 = _pack_edges(edge_index)
    part0 = _seg_sum(h0, sd0)
    t0, s1a, s2a = _layer_a(part0[0], part0[1], h0, Wrel0, Wroot0,
                            brel0.reshape(1, D))
    h1 = _layer_b(t0, s1a, s2a, gamma0, beta0, relu=True)

    sdf = _pack_edges(edge_index_full)
    part1 = _seg_sum(h1, sdf)
    t1, s1b, s2b = _layer_a(part1[0], part1[1], h1, Wrel1_2h, Wroot1_2h,
                            brel1_2h.reshape(1, D))
    h2 = _layer_b(t1, s1b, s2b, gamma1, beta1, relu=False)
    return h2


# speedup vs baseline: 1.2726x; 1.2726x over previous
import functools

import jax
import jax.numpy as jnp
from jax import lax
from jax.experimental import pallas as pl
from jax.experimental.pallas import tpu as pltpu
from jax.experimental.pallas import tpu_sc as plsc

N = 10000
D = 128
E = 320000

NC = 2
NS = 16
NW = NC * NS
K = 128
G = 8
NCHF = 136
NCHS = 24
NGF = NCHF // G
NGS = NCHS // G
E_PAD = NS * (NCHF + NCHS) * K
N_PAD = 10112
ROWS_PER_TILE = N_PAD // NS

_BLK = 2000


def _seg_sum_body(h_hbm, sd_hbm, out_hbm,
                  ib0, ib1, buf0, buf1, isem0, isem1, dsem0, dsem1, acc):
    c = lax.axis_index("c")
    s = lax.axis_index("s")
    wid = c * NS + s
    ng = jnp.where(c == 0, NGF, NGS)

    @pl.when(ng > 0)
    def _():
        pltpu.async_copy(sd_hbm.at[wid, pl.ds(0, G)], ib0, isem0)

    zero = jnp.zeros((16,), jnp.float32)

    def _zrow(i, carry):
        for l in range(D // 16):
            buf0[i, pl.ds(l * 16, 16)] = zero
        return carry

    lax.fori_loop(0, K, _zrow, 0)
    for z in range(ROWS_PER_TILE // K):
        pltpu.sync_copy(buf0, acc.at[pl.ds(s * ROWS_PER_TILE + z * K, K)])
    _rem = ROWS_PER_TILE % K
    if _rem:
        pltpu.sync_copy(
            buf0.at[pl.ds(0, _rem)],
            acc.at[pl.ds(s * ROWS_PER_TILE + (ROWS_PER_TILE // K) * K, _rem)])
    plsc.subcore_barrier()

    def _process(g, ib_cur, isem_cur, ib_next, isem_next):
        pltpu.make_async_copy(sd_hbm.at[wid, pl.ds(0, G)], ib_cur,
                              isem_cur).wait()

        @pl.when(g + 1 < ng)
        def _():
            pltpu.async_copy(sd_hbm.at[wid, pl.ds((g + 1) * G, G)],
                             ib_next, isem_next)

        pltpu.async_copy(h_hbm.at[ib_cur.at[0, 0]], buf0, dsem0)
        pltpu.async_copy(h_hbm.at[ib_cur.at[1, 0]], buf1, dsem1)
        for i in range(G):
            buf, dsem = (buf0, dsem0) if i % 2 == 0 else (buf1, dsem1)
            pltpu.make_async_copy(h_hbm.at[ib_cur.at[i, 0]], buf,
                                  dsem).wait()
            pltpu.sync_copy(buf, acc.at[ib_cur.at[i, 1]], add=True)
            if i + 2 < G:
                pltpu.async_copy(h_hbm.at[ib_cur.at[i + 2, 0]], buf, dsem)

    def _body(g, carry):
        @pl.when(g % 2 == 0)
        def _():
            _process(g, ib0, isem0, ib1, isem1)

        @pl.when(g % 2 == 1)
        def _():
            _process(g, ib1, isem1, ib0, isem0)

        return carry

    lax.fori_loop(0, ng, _body, 0)
    plsc.subcore_barrier()

    pltpu.sync_copy(acc.at[pl.ds(s * ROWS_PER_TILE, ROWS_PER_TILE)],
                    out_hbm.at[c, pl.ds(s * ROWS_PER_TILE, ROWS_PER_TILE)])


_seg_sum = functools.partial(
    pl.kernel,
    out_type=jax.ShapeDtypeStruct((NC, N_PAD, D), jnp.float32),
    mesh=plsc.VectorSubcoreMesh(core_axis_name="c", subcore_axis_name="s"),
    scratch_types=[
        pltpu.VMEM((G, 2, K), jnp.int32),
        pltpu.VMEM((G, 2, K), jnp.int32),
        pltpu.VMEM((K, D), jnp.float32),
        pltpu.VMEM((K, D), jnp.float32),
        pltpu.SemaphoreType.DMA,
        pltpu.SemaphoreType.DMA,
        pltpu.SemaphoreType.DMA,
        pltpu.SemaphoreType.DMA,
        pltpu.VMEM_SHARED((N_PAD, D), jnp.float32),
    ],
)(_seg_sum_body)


def _pack_edges(edge_index):
    pad = E_PAD - E
    src = jnp.concatenate([edge_index[0], jnp.zeros((pad,), jnp.int32)])
    dst = jnp.concatenate([edge_index[1], jnp.full((pad,), N, jnp.int32)])
    sd = jnp.stack([src.reshape(-1, K), dst.reshape(-1, K)], axis=1)
    nf = NS * NCHF
    fast = sd[:nf].reshape(NS, NCHF, 2, K)
    slow = sd[nf:].reshape(NS, NCHS, 2, K)
    slow = jnp.pad(slow, ((0, 0), (0, NCHF - NCHS), (0, 0), (0, 0)))
    return jnp.concatenate([fast, slow], axis=0)


def _dense0_body(x_ref, w_ref, b_ref, o_ref):
    o_ref[...] = (jnp.dot(x_ref[...], w_ref[...],
                          preferred_element_type=jnp.float32) + b_ref[...])


def _dense0(x, Wx, bx):
    return pl.pallas_call(
        _dense0_body,
        grid=(N // _BLK,),
        in_specs=[
            pl.BlockSpec((_BLK, D), lambda i: (i, 0)),
            pl.BlockSpec((D, D), lambda i: (0, 0)),
            pl.BlockSpec((1, D), lambda i: (0, 0)),
        ],
        out_specs=pl.BlockSpec((_BLK, D), lambda i: (i, 0)),
        out_shape=jax.ShapeDtypeStruct((N, D), jnp.float32),
    )(x, Wx, bx.reshape(1, D))


def _layer_a_body(p0_ref, p1_ref, h_ref, wrel_ref, wroot_ref, brel_ref,
                  t_ref, s1_ref, s2_ref):
    agg = p0_ref[...] + p1_ref[...]
    h = h_ref[...]
    t = (jnp.dot(agg, wrel_ref[...], preferred_element_type=jnp.float32)
         + jnp.dot(h, wroot_ref[...], preferred_element_type=jnp.float32)
         + h + brel_ref[...])
    t_ref[...] = t

    @pl.when(pl.program_id(0) == 0)
    def _():
        s1_ref[...] = jnp.zeros_like(s1_ref)
        s2_ref[...] = jnp.zeros_like(s2_ref)

    s1_ref[...] += jnp.broadcast_to(jnp.sum(t, axis=0), (8, D))
    s2_ref[...] += jnp.broadcast_to(jnp.sum(t * t, axis=0), (8, D))


def _layer_a(p0, p1, h, wrel, wroot, brel):
    return pl.pallas_call(
        _layer_a_body,
        grid=(N // _BLK,),
        in_specs=[
            pl.BlockSpec((_BLK, D), lambda i: (i, 0)),
            pl.BlockSpec((_BLK, D), lambda i: (i, 0)),
            pl.BlockSpec((_BLK, D), lambda i: (i, 0)),
            pl.BlockSpec((D, D), lambda i: (0, 0)),
            pl.BlockSpec((D, D), lambda i: (0, 0)),
            pl.BlockSpec((1, D), lambda i: (0, 0)),
        ],
        out_specs=[
            pl.BlockSpec((_BLK, D), lambda i: (i, 0)),
            pl.BlockSpec((8, D), lambda i: (0, 0)),
            pl.BlockSpec((8, D), lambda i: (0, 0)),
        ],
        out_shape=[
            jax.ShapeDtypeStruct((N, D), jnp.float32),
            jax.ShapeDtypeStruct((8, D), jnp.float32),
            jax.ShapeDtypeStruct((8, D), jnp.float32),
        ],
    )(p0, p1, h, wrel, wroot, brel)


def _layer_b_body(t_ref, s1_ref, s2_ref, g_ref, bb_ref, o_ref, *, relu):
    mu = s1_ref[0:1, :] * (1.0 / N)
    var = s2_ref[0:1, :] * (1.0 / N) - mu * mu
    inv = lax.rsqrt(var + 1e-5)
    o = g_ref[...] * ((t_ref[...] - mu) * inv) + bb_ref[...]
    if relu:
        o = jnp.maximum(o, 0.0)
    o_ref[...] = o


def _layer_b(t, s1, s2, gamma, beta, relu):
    return pl.pallas_call(
        functools.partial(_layer_b_body, relu=relu),
        grid=(N // _BLK,),
        in_specs=[
            pl.BlockSpec((_BLK, D), lambda i: (i, 0)),
            pl.BlockSpec((8, D), lambda i: (0, 0)),
            pl.BlockSpec((8, D), lambda i: (0, 0)),
            pl.BlockSpec((1, D), lambda i: (0, 0)),
            pl.BlockSpec((1, D), lambda i: (0, 0)),
        ],
        out_specs=pl.BlockSpec((_BLK, D), lambda i: (i, 0)),
        out_shape=jax.ShapeDtypeStruct((N, D), jnp.float32),
    )(t, s1, s2, gamma.reshape(1, D), beta.reshape(1, D))


def kernel(x, edge_index, edge_attr, edge_index_full, Wx, bx,
           Wrel0, brel0, Wroot0, Wrel1, brel1, Wroot1,
           Wrel1_2h, brel1_2h, Wroot1_2h, gamma0, beta0, gamma1, beta1):
    h0 = _dense0(x, Wx, bx)

    sd0 = _pack_edges(edge_index)
    part0 = _seg_sum(h0, sd0)
    t0, s1a, s2a = _layer_a(part0[0], part0[1], h0, Wrel0, Wroot0,
                            brel0.reshape(1, D))
    h1 = _layer_b(t0, s1a, s2a, gamma0, beta0, relu=True)

    sdf = _pack_edges(edge_index_full)
    part1 = _seg_sum(h1, sdf)
    t1, s1b, s2b = _layer_a(part1[0], part1[1], h1, Wrel1_2h, Wroot1_2h,
                            brel1_2h.reshape(1, D))
    h2 = _layer_b(t1, s1b, s2b, gamma1, beta1, relu=False)
    return h2


# speedup vs baseline: 1.3600x; 1.0687x over previous
import functools

import jax
import jax.numpy as jnp
from jax import lax
from jax.experimental import pallas as pl
from jax.experimental.pallas import tpu as pltpu
from jax.experimental.pallas import tpu_sc as plsc

N = 10000
D = 128
E = 320000

NC = 2
NS = 16
NW = NC * NS
K = 128
G = 8
NCHF = 112
NCHS = 48
NGF = NCHF // G
NGS = NCHS // G
E_PAD = NS * (NCHF + NCHS) * K
N_PAD = 10112
ROWS_PER_TILE = N_PAD // NS

_BLK = 2000


def _seg_sum_body(h_hbm, sd_hbm, out_hbm,
                  ib0, ib1, buf0, buf1, isem0, isem1, dsem0, dsem1, acc):
    c = lax.axis_index("c")
    s = lax.axis_index("s")
    wid = c * NS + s
    ng = jnp.where(c == 0, NGF, NGS)

    @pl.when(ng > 0)
    def _():
        pltpu.async_copy(sd_hbm.at[wid, pl.ds(0, G)], ib0, isem0)

    zero = jnp.zeros((16,), jnp.float32)

    def _zrow(i, carry):
        for l in range(D // 16):
            buf0[i, pl.ds(l * 16, 16)] = zero
        return carry

    lax.fori_loop(0, K, _zrow, 0)
    for z in range(ROWS_PER_TILE // K):
        pltpu.sync_copy(buf0, acc.at[pl.ds(s * ROWS_PER_TILE + z * K, K)])
    _rem = ROWS_PER_TILE % K
    if _rem:
        pltpu.sync_copy(
            buf0.at[pl.ds(0, _rem)],
            acc.at[pl.ds(s * ROWS_PER_TILE + (ROWS_PER_TILE // K) * K, _rem)])
    plsc.subcore_barrier()

    def _process(g, ib_cur, isem_cur, ib_next, isem_next):
        pltpu.make_async_copy(sd_hbm.at[wid, pl.ds(0, G)], ib_cur,
                              isem_cur).wait()

        @pl.when(g + 1 < ng)
        def _():
            pltpu.async_copy(sd_hbm.at[wid, pl.ds((g + 1) * G, G)],
                             ib_next, isem_next)

        pltpu.async_copy(h_hbm.at[ib_cur.at[0, 0]], buf0, dsem0)
        pltpu.async_copy(h_hbm.at[ib_cur.at[1, 0]], buf1, dsem1)
        for i in range(G):
            buf, dsem = (buf0, dsem0) if i % 2 == 0 else (buf1, dsem1)
            pltpu.make_async_copy(h_hbm.at[ib_cur.at[i, 0]], buf,
                                  dsem).wait()
            pltpu.sync_copy(buf, acc.at[ib_cur.at[i, 1]], add=True)
            if i + 2 < G:
                pltpu.async_copy(h_hbm.at[ib_cur.at[i + 2, 0]], buf, dsem)

    def _body(g, carry):
        @pl.when(g % 2 == 0)
        def _():
            _process(g, ib0, isem0, ib1, isem1)

        @pl.when(g % 2 == 1)
        def _():
            _process(g, ib1, isem1, ib0, isem0)

        return carry

    lax.fori_loop(0, ng, _body, 0)
    plsc.subcore_barrier()

    pltpu.sync_copy(acc.at[pl.ds(s * ROWS_PER_TILE, ROWS_PER_TILE)],
                    out_hbm.at[c, pl.ds(s * ROWS_PER_TILE, ROWS_PER_TILE)])


_seg_sum = functools.partial(
    pl.kernel,
    out_type=jax.ShapeDtypeStruct((NC, N_PAD, D), jnp.float32),
    mesh=plsc.VectorSubcoreMesh(core_axis_name="c", subcore_axis_name="s"),
    scratch_types=[
        pltpu.VMEM((G, 2, K), jnp.int32),
        pltpu.VMEM((G, 2, K), jnp.int32),
        pltpu.VMEM((K, D), jnp.float32),
        pltpu.VMEM((K, D), jnp.float32),
        pltpu.SemaphoreType.DMA,
        pltpu.SemaphoreType.DMA,
        pltpu.SemaphoreType.DMA,
        pltpu.SemaphoreType.DMA,
        pltpu.VMEM_SHARED((N_PAD, D), jnp.float32),
    ],
)(_seg_sum_body)


def _pack_edges(edge_index):
    pad = E_PAD - E
    src = jnp.concatenate([edge_index[0], jnp.zeros((pad,), jnp.int32)])
    dst = jnp.concatenate([edge_index[1], jnp.full((pad,), N, jnp.int32)])
    sd = jnp.stack([src.reshape(-1, K), dst.reshape(-1, K)], axis=1)
    nf = NS * NCHF
    fast = sd[:nf].reshape(NS, NCHF, 2, K)
    slow = sd[nf:].reshape(NS, NCHS, 2, K)
    slow = jnp.pad(slow, ((0, 0), (0, NCHF - NCHS), (0, 0), (0, 0)))
    return jnp.concatenate([fast, slow], axis=0)


def _dense0_body(x_ref, w_ref, b_ref, o_ref):
    o_ref[...] = (jnp.dot(x_ref[...], w_ref[...],
                          preferred_element_type=jnp.float32) + b_ref[...])


def _dense0(x, Wx, bx):
    return pl.pallas_call(
        _dense0_body,
        grid=(N // _BLK,),
        in_specs=[
            pl.BlockSpec((_BLK, D), lambda i: (i, 0)),
            pl.BlockSpec((D, D), lambda i: (0, 0)),
            pl.BlockSpec((1, D), lambda i: (0, 0)),
        ],
        out_specs=pl.BlockSpec((_BLK, D), lambda i: (i, 0)),
        out_shape=jax.ShapeDtypeStruct((N, D), jnp.float32),
    )(x, Wx, bx.reshape(1, D))


def _layer_a_body(p0_ref, p1_ref, h_ref, wrel_ref, wroot_ref, brel_ref,
                  t_ref, s1_ref, s2_ref):
    agg = p0_ref[...] + p1_ref[...]
    h = h_ref[...]
    t = (jnp.dot(agg, wrel_ref[...], preferred_element_type=jnp.float32)
         + jnp.dot(h, wroot_ref[...], preferred_element_type=jnp.float32)
         + h + brel_ref[...])
    t_ref[...] = t

    @pl.when(pl.program_id(0) == 0)
    def _():
        s1_ref[...] = jnp.zeros_like(s1_ref)
        s2_ref[...] = jnp.zeros_like(s2_ref)

    s1_ref[...] += jnp.broadcast_to(jnp.sum(t, axis=0), (8, D))
    s2_ref[...] += jnp.broadcast_to(jnp.sum(t * t, axis=0), (8, D))


def _layer_a(p0, p1, h, wrel, wroot, brel):
    return pl.pallas_call(
        _layer_a_body,
        grid=(N // _BLK,),
        in_specs=[
            pl.BlockSpec((_BLK, D), lambda i: (i, 0)),
            pl.BlockSpec((_BLK, D), lambda i: (i, 0)),
            pl.BlockSpec((_BLK, D), lambda i: (i, 0)),
            pl.BlockSpec((D, D), lambda i: (0, 0)),
            pl.BlockSpec((D, D), lambda i: (0, 0)),
            pl.BlockSpec((1, D), lambda i: (0, 0)),
        ],
        out_specs=[
            pl.BlockSpec((_BLK, D), lambda i: (i, 0)),
            pl.BlockSpec((8, D), lambda i: (0, 0)),
            pl.BlockSpec((8, D), lambda i: (0, 0)),
        ],
        out_shape=[
            jax.ShapeDtypeStruct((N, D), jnp.float32),
            jax.ShapeDtypeStruct((8, D), jnp.float32),
            jax.ShapeDtypeStruct((8, D), jnp.float32),
        ],
    )(p0, p1, h, wrel, wroot, brel)


def _layer_b_body(t_ref, s1_ref, s2_ref, g_ref, bb_ref, o_ref, *, relu):
    mu = s1_ref[0:1, :] * (1.0 / N)
    var = s2_ref[0:1, :] * (1.0 / N) - mu * mu
    inv = lax.rsqrt(var + 1e-5)
    o = g_ref[...] * ((t_ref[...] - mu) * inv) + bb_ref[...]
    if relu:
        o = jnp.maximum(o, 0.0)
    o_ref[...] = o


def _layer_b(t, s1, s2, gamma, beta, relu):
    return pl.pallas_call(
        functools.partial(_layer_b_body, relu=relu),
        grid=(N // _BLK,),
        in_specs=[
            pl.BlockSpec((_BLK, D), lambda i: (i, 0)),
            pl.BlockSpec((8, D), lambda i: (0, 0)),
            pl.BlockSpec((8, D), lambda i: (0, 0)),
            pl.BlockSpec((1, D), lambda i: (0, 0)),
            pl.BlockSpec((1, D), lambda i: (0, 0)),
        ],
        out_specs=pl.BlockSpec((_BLK, D), lambda i: (i, 0)),
        out_shape=jax.ShapeDtypeStruct((N, D), jnp.float32),
    )(t, s1, s2, gamma.reshape(1, D), beta.reshape(1, D))


def kernel(x, edge_index, edge_attr, edge_index_full, Wx, bx,
           Wrel0, brel0, Wroot0, Wrel1, brel1, Wroot1,
           Wrel1_2h, brel1_2h, Wroot1_2h, gamma0, beta0, gamma1, beta1):
    h0 = _dense0(x, Wx, bx)

    sd0 = _pack_edges(edge_index)
    part0 = _seg_sum(h0, sd0)
    t0, s1a, s2a = _layer_a(part0[0], part0[1], h0, Wrel0, Wroot0,
                            brel0.reshape(1, D))
    h1 = _layer_b(t0, s1a, s2a, gamma0, beta0, relu=True)

    sdf = _pack_edges(edge_index_full)
    part1 = _seg_sum(h1, sdf)
    t1, s1b, s2b = _layer_a(part1[0], part1[1], h1, Wrel1_2h, Wroot1_2h,
                            brel1_2h.reshape(1, D))
    h2 = _layer_b(t1, s1b, s2b, gamma1, beta1, relu=False)
    return h2
